# baseline (device time: 11457 ns/iter reference)
import jax
import jax.numpy as jnp
from jax import lax
from jax.experimental import pallas as pl
from jax.experimental.pallas import tpu as pltpu

SIZES = (64, 64, 48, 32, 16, 16, 16)
C = len(SIZES)
OFFS = tuple(sum(SIZES[:i]) for i in range(C))


def kernel(x):
    m, n = x.shape
    half = m // 2
    assert sum(SIZES) == half

    def body(x_hbm, out_hbm, x_vmem, send_buf, yrecv_buf, sum_buf, xrecv_buf,
             in_sems, ysend_sems, yrecv_sems, xsend_sems, xrecv_sems,
             outh_sems, outo_sems):
        my_x = lax.axis_index("x")
        my_y = lax.axis_index("y")
        my_z = lax.axis_index("z")
        ypartner = (my_x, 1 - my_y, my_z)
        xpartner = (1 - my_x, my_y, my_z)
        my_off = my_x * half
        other_off = (1 - my_x) * half

        fetches = []
        for i, (o, s) in enumerate(zip(OFFS, SIZES)):
            cp = pltpu.make_async_copy(
                x_hbm.at[pl.ds(my_off + o, s), :],
                x_vmem.at[pl.ds(o, s), :],
                in_sems.at[i],
            )
            cp.start()
            fetches.append(cp)

        barrier_sem = pltpu.get_barrier_semaphore()
        for nbr in (ypartner, xpartner):
            pl.semaphore_signal(
                barrier_sem, inc=1,
                device_id=nbr, device_id_type=pl.DeviceIdType.MESH,
            )
        pl.semaphore_wait(barrier_sem, 2)

        yrdmas = []
        for i, (o, s) in enumerate(zip(OFFS, SIZES)):
            fetches[i].wait()
            send_buf[pl.ds(o, s), :] = (
                x_vmem[pl.ds(o, s), :].astype(jnp.bfloat16)
            )
            rdma = pltpu.make_async_remote_copy(
                src_ref=send_buf.at[pl.ds(o, s), :],
                dst_ref=yrecv_buf.at[pl.ds(o, s), :],
                send_sem=ysend_sems.at[i],
                recv_sem=yrecv_sems.at[i],
                device_id=ypartner,
                device_id_type=pl.DeviceIdType.MESH,
            )
            rdma.start()
            yrdmas.append(rdma)

        xrdmas = []
        hstores = []
        for i, (o, s) in enumerate(zip(OFFS, SIZES)):
            yrdmas[i].wait_recv()
            sum_buf[pl.ds(o, s), :] = (
                x_vmem[pl.ds(o, s), :]
                + yrecv_buf[pl.ds(o, s), :].astype(jnp.float32)
            ).astype(jnp.bfloat16)
            rdma = pltpu.make_async_remote_copy(
                src_ref=sum_buf.at[pl.ds(o, s), :],
                dst_ref=xrecv_buf.at[pl.ds(o, s), :],
                send_sem=xsend_sems.at[i],
                recv_sem=xrecv_sems.at[i],
                device_id=xpartner,
                device_id_type=pl.DeviceIdType.MESH,
            )
            rdma.start()
            xrdmas.append(rdma)
            st = pltpu.make_async_copy(
                sum_buf.at[pl.ds(o, s), :],
                out_hbm.at[pl.ds(my_off + o, s), :],
                outh_sems.at[i],
            )
            st.start()
            hstores.append(st)

        ostores = []
        for i, (o, s) in enumerate(zip(OFFS, SIZES)):
            xrdmas[i].wait_recv()
            st = pltpu.make_async_copy(
                xrecv_buf.at[pl.ds(o, s), :],
                out_hbm.at[pl.ds(other_off + o, s), :],
                outo_sems.at[i],
            )
            st.start()
            ostores.append(st)

        for i in range(C):
            hstores[i].wait()
            ostores[i].wait()
            yrdmas[i].wait_send()
            xrdmas[i].wait_send()

    x = pltpu.with_memory_space_constraint(x, pltpu.MemorySpace.HBM)
    return pl.pallas_call(
        body,
        out_shape=jax.ShapeDtypeStruct((m, n), jnp.bfloat16),
        in_specs=[pl.BlockSpec(memory_space=pltpu.MemorySpace.HBM)],
        out_specs=pl.BlockSpec(memory_space=pltpu.MemorySpace.HBM),
        scratch_shapes=[
            pltpu.VMEM((half, n), jnp.float32),
            pltpu.VMEM((half, n), jnp.bfloat16),
            pltpu.VMEM((half, n), jnp.bfloat16),
            pltpu.VMEM((half, n), jnp.bfloat16),
            pltpu.VMEM((half, n), jnp.bfloat16),
            pltpu.SemaphoreType.DMA((C,)),
            pltpu.SemaphoreType.DMA((C,)),
            pltpu.SemaphoreType.DMA((C,)),
            pltpu.SemaphoreType.DMA((C,)),
            pltpu.SemaphoreType.DMA((C,)),
            pltpu.SemaphoreType.DMA((C,)),
            pltpu.SemaphoreType.DMA((C,)),
        ],
        compiler_params=pltpu.CompilerParams(collective_id=0),
    )(x)


# device time: 11133 ns/iter; 1.0291x vs baseline; 1.0291x over previous
import jax
import jax.numpy as jnp
from jax import lax
from jax.experimental import pallas as pl
from jax.experimental.pallas import tpu as pltpu

SIZES = (32, 32, 32, 32, 32, 32, 32, 32)
C = len(SIZES)
OFFS = tuple(sum(SIZES[:i]) for i in range(C))


def kernel(x):
    m, n = x.shape
    half = m // 2
    assert sum(SIZES) == half

    def body(x_hbm, out_hbm, x_vmem, send_buf, yrecv_buf, sum_buf, xrecv_buf,
             in_sems, ysend_sems, yrecv_sems, xsend_sems, xrecv_sems,
             outh_sems, outo_sems):
        my_x = lax.axis_index("x")
        my_y = lax.axis_index("y")
        my_z = lax.axis_index("z")
        ypartner = (my_x, 1 - my_y, my_z)
        xpartner = (1 - my_x, my_y, my_z)
        my_off = my_x * half
        other_off = (1 - my_x) * half

        fetches = []
        for i, (o, s) in enumerate(zip(OFFS, SIZES)):
            cp = pltpu.make_async_copy(
                x_hbm.at[pl.ds(my_off + o, s), :],
                x_vmem.at[pl.ds(o, s), :],
                in_sems.at[i],
            )
            cp.start()
            fetches.append(cp)

        barrier_sem = pltpu.get_barrier_semaphore()
        for nbr in (ypartner, xpartner):
            pl.semaphore_signal(
                barrier_sem, inc=1,
                device_id=nbr, device_id_type=pl.DeviceIdType.MESH,
            )
        pl.semaphore_wait(barrier_sem, 2)

        yrdmas = []
        for i, (o, s) in enumerate(zip(OFFS, SIZES)):
            fetches[i].wait()
            send_buf[pl.ds(o, s), :] = (
                x_vmem[pl.ds(o, s), :].astype(jnp.bfloat16)
            )
            rdma = pltpu.make_async_remote_copy(
                src_ref=send_buf.at[pl.ds(o, s), :],
                dst_ref=yrecv_buf.at[pl.ds(o, s), :],
                send_sem=ysend_sems.at[i],
                recv_sem=yrecv_sems.at[i],
                device_id=ypartner,
                device_id_type=pl.DeviceIdType.MESH,
            )
            rdma.start()
            yrdmas.append(rdma)

        xrdmas = []
        hstores = []
        for i, (o, s) in enumerate(zip(OFFS, SIZES)):
            yrdmas[i].wait_recv()
            sum_buf[pl.ds(o, s), :] = (
                x_vmem[pl.ds(o, s), :]
                + yrecv_buf[pl.ds(o, s), :].astype(jnp.float32)
            ).astype(jnp.bfloat16)
            rdma = pltpu.make_async_remote_copy(
                src_ref=sum_buf.at[pl.ds(o, s), :],
                dst_ref=xrecv_buf.at[pl.ds(o, s), :],
                send_sem=xsend_sems.at[i],
                recv_sem=xrecv_sems.at[i],
                device_id=xpartner,
                device_id_type=pl.DeviceIdType.MESH,
            )
            rdma.start()
            xrdmas.append(rdma)
            st = pltpu.make_async_copy(
                sum_buf.at[pl.ds(o, s), :],
                out_hbm.at[pl.ds(my_off + o, s), :],
                outh_sems.at[i],
            )
            st.start()
            hstores.append(st)

        ostores = []
        for i, (o, s) in enumerate(zip(OFFS, SIZES)):
            xrdmas[i].wait_recv()
            st = pltpu.make_async_copy(
                xrecv_buf.at[pl.ds(o, s), :],
                out_hbm.at[pl.ds(other_off + o, s), :],
                outo_sems.at[i],
            )
            st.start()
            ostores.append(st)

        for i in range(C):
            hstores[i].wait()
            ostores[i].wait()
            yrdmas[i].wait_send()
            xrdmas[i].wait_send()

    x = pltpu.with_memory_space_constraint(x, pltpu.MemorySpace.HBM)
    return pl.pallas_call(
        body,
        out_shape=jax.ShapeDtypeStruct((m, n), jnp.bfloat16),
        in_specs=[pl.BlockSpec(memory_space=pltpu.MemorySpace.HBM)],
        out_specs=pl.BlockSpec(memory_space=pltpu.MemorySpace.HBM),
        scratch_shapes=[
            pltpu.VMEM((half, n), jnp.float32),
            pltpu.VMEM((half, n), jnp.bfloat16),
            pltpu.VMEM((half, n), jnp.bfloat16),
            pltpu.VMEM((half, n), jnp.bfloat16),
            pltpu.VMEM((half, n), jnp.bfloat16),
            pltpu.SemaphoreType.DMA((C,)),
            pltpu.SemaphoreType.DMA((C,)),
            pltpu.SemaphoreType.DMA((C,)),
            pltpu.SemaphoreType.DMA((C,)),
            pltpu.SemaphoreType.DMA((C,)),
            pltpu.SemaphoreType.DMA((C,)),
            pltpu.SemaphoreType.DMA((C,)),
        ],
        compiler_params=pltpu.CompilerParams(collective_id=0),
    )(x)
